# baseline (speedup 1.0000x reference)
import functools

import jax
import jax.numpy as jnp
from jax import lax
from jax.experimental import pallas as pl
from jax.experimental.pallas import tpu as pltpu
from jax.experimental.pallas import tpu_sc as plsc

_H = 2048
_W = 2048
_NBOX = 100000
_NW = 16
_NB = 256
_NBH = _NB // 2
_CAP = 66
_SEG = 200
_CVEC = 26
_NCHUNK = 15

_O_CHUNK = 0
_O_CURSOR = 1664
_O_STAGE = 1792
_O_CNTS = 0
_O_SEGB = 4096
_BLOB = 27392


def _sc_body(boxes, heat, size, blob, planes, sseg, scnt, sem):
    wid = lax.axis_index("s")
    lane = lax.iota(jnp.int32, 16)
    lane4 = lane * 4
    zero16 = jnp.zeros((16,), jnp.float32)
    one16 = jnp.ones((16,), jnp.float32)

    chunk = blob.at[pl.ds(_O_CHUNK, _CVEC * 64)]
    cursor = blob.at[pl.ds(_O_CURSOR, _NBH)]
    stage = blob.at[pl.ds(_O_STAGE, _NBH * _SEG)]
    cntsv = blob.at[pl.ds(_O_CNTS, _NW * _NB)]
    segbuf = blob.at[pl.ds(_O_SEGB, _NW * _SEG)]

    start_v = wid * 390 + jnp.minimum(wid, 10)
    has_tail = wid < 10

    for rnd in range(2):
        def zero_cursor(i, _):
            cursor[pl.ds(i * 16, 16)] = zero16
            return 0

        lax.fori_loop(0, _NBH // 16, zero_cursor, 0)

        def route_vec(vlocal, rnd=rnd):
            base = vlocal * 64 + lane4
            x = plsc.load_gather(chunk, [base])
            y = plsc.load_gather(chunk, [base + 1])
            wi = plsc.load_gather(chunk, [base + 2])
            hi = plsc.load_gather(chunk, [base + 3])
            cx = (x * _W).astype(jnp.int32)
            cy = (y * _H).astype(jnp.int32)
            bucket = cy >> 3
            m = (bucket >> 7) == rnd
            bloc = bucket & (_NBH - 1)
            rel = (cy & 7) * _W + cx
            rank, lastm = plsc.scan_count(bloc, mask=m)
            basec = plsc.bitcast(plsc.load_gather(cursor, [bloc]), jnp.int32)
            pos = jnp.minimum(basec + rank - 1, _CAP - 1)
            addr = bloc * _SEG + pos * 3
            plsc.store_scatter(stage, [addr],
                               plsc.bitcast(rel, jnp.float32), mask=m)
            plsc.store_scatter(stage, [addr + 1], wi, mask=m)
            plsc.store_scatter(stage, [addr + 2], hi, mask=m)
            plsc.store_scatter(cursor, [bloc],
                               plsc.bitcast(pos + 1, jnp.float32), mask=lastm)

        def chunk_body(c, _):
            cs = start_v + c * _CVEC
            pltpu.sync_copy(boxes.at[pl.ds(cs * 64, _CVEC * 64)], chunk)

            def vec_body(vl, _):
                route_vec(vl)
                return 0

            lax.fori_loop(0, _CVEC, vec_body, 0)
            return 0

        lax.fori_loop(0, _NCHUNK, chunk_body, 0)

        @pl.when(has_tail)
        def _tail():
            ts = start_v + _NCHUNK * _CVEC
            pltpu.sync_copy(boxes.at[pl.ds(ts * 64, 64)],
                            blob.at[pl.ds(_O_CHUNK, 64)])
            route_vec(0)

        def ship(bb, _, rnd=rnd):
            b = rnd * _NBH + bb
            pltpu.async_copy(
                blob.at[pl.ds(_O_STAGE + bb * _SEG, _SEG)],
                sseg.at[pl.ds((b * _NW + wid) * _SEG, _SEG)],
                sem,
            )
            return 0

        lax.fori_loop(0, _NBH, ship, 0)

        def drain(bb, _):
            pltpu.make_async_copy(
                blob.at[pl.ds(_O_STAGE, _SEG)],
                sseg.at[pl.ds(wid * _SEG, _SEG)],
                sem,
            ).wait()
            return 0

        lax.fori_loop(0, _NBH, drain, 0)
        pltpu.sync_copy(cursor,
                        scnt.at[pl.ds(wid * _NB + rnd * _NBH, _NBH)])

    plsc.subcore_barrier()

    pltpu.sync_copy(scnt, cntsv)

    def zero_planes(i, _):
        planes[i >> 7, pl.ds((i & 127) * 16, 16)] = zero16
        return 0

    lax.fori_loop(0, 24 * 128, zero_planes, 0)

    def bucket_body(k, _):
        b = wid * 16 + k
        pltpu.sync_copy(sseg.at[pl.ds(b * _NW * _SEG, _NW * _SEG)], segbuf)
        cbvec = plsc.bitcast(
            plsc.load_gather(cntsv, [lane * _NB + b]), jnp.int32)

        def seg_pass(value_sel):
            for s in range(_NW):
                cs = jnp.sum(jnp.where(lane == s, cbvec, 0))
                nvs = (cs + 15) >> 4

                def seg_vec(v, _):
                    j = v * 16 + lane
                    idx = s * _SEG + j * 3
                    rel = plsc.bitcast(
                        plsc.load_gather(segbuf, [idx]), jnp.int32)
                    r = rel >> 11
                    c = rel & (_W - 1)
                    valm = j < cs
                    if value_sel == 0:
                        wv = plsc.load_gather(segbuf, [idx + 1])
                        hv = plsc.load_gather(segbuf, [idx + 2])
                        plsc.store_scatter(planes, [r, c], one16, mask=valm)
                        plsc.store_scatter(planes, [r + 8, c], wv, mask=valm)
                        plsc.store_scatter(planes, [r + 16, c], hv,
                                           mask=valm)
                    else:
                        plsc.store_scatter(planes, [r, c], zero16, mask=valm)
                        plsc.store_scatter(planes, [r + 8, c], zero16,
                                           mask=valm)
                        plsc.store_scatter(planes, [r + 16, c], zero16,
                                           mask=valm)
                    return 0

                lax.fori_loop(0, nvs, seg_vec, 0)

        seg_pass(0)
        r0 = b * 8
        pltpu.sync_copy(planes.at[pl.ds(0, 8), :],
                        heat.at[pl.ds(r0, 8), :])
        pltpu.sync_copy(planes.at[pl.ds(8, 8), :],
                        size.at[0, pl.ds(r0, 8), :])
        pltpu.sync_copy(planes.at[pl.ds(16, 8), :],
                        size.at[1, pl.ds(r0, 8), :])
        seg_pass(1)
        return 0

    lax.fori_loop(0, 16, bucket_body, 0)


_sc_kernel = functools.partial(
    pl.kernel,
    out_type=(
        jax.ShapeDtypeStruct((_H, _W), jnp.float32),
        jax.ShapeDtypeStruct((2, _H, _W), jnp.float32),
    ),
    mesh=plsc.VectorSubcoreMesh(
        core_axis_name="c", subcore_axis_name="s",
        num_cores=1, num_subcores=_NW,
    ),
    compiler_params=pltpu.CompilerParams(needs_layout_passes=False, skip_device_barrier=True),
    scratch_types=[
        pltpu.VMEM((_BLOB,), jnp.float32),
        pltpu.VMEM((24, _W), jnp.float32),
        pltpu.VMEM_SHARED((_NB * _NW * _SEG,), jnp.float32),
        pltpu.VMEM_SHARED((_NW * _NB,), jnp.float32),
        pltpu.SemaphoreType.DMA,
    ],
)(_sc_body)


def kernel(boxes):
    heat, size = _sc_kernel(boxes.reshape(-1))
    return heat[None, None], size[None]


# speedup vs baseline: 1.0780x; 1.0780x over previous
import functools

import jax
import jax.numpy as jnp
from jax import lax
from jax.experimental import pallas as pl
from jax.experimental.pallas import tpu as pltpu
from jax.experimental.pallas import tpu_sc as plsc

_H = 2048
_W = 2048
_NBOX = 100000
_NW = 16
_NB = 256
_NBH = _NB // 2
_CAP = 66
_SEG = 200
_CVEC = 26
_NCHUNK = 15

_O_CHUNK = 0
_O_CURSOR = 1664
_O_STAGE = 1792
_O_CNTS = 0
_O_SEGB = 4096
_BLOB = 27392


def _sc_body(xs, ys, ws, hs, heat, size, blob, planes, sseg, scnt, sem):
    wid = lax.axis_index("s")
    lane = lax.iota(jnp.int32, 16)
    zero16i = jnp.zeros((16,), jnp.int32)
    zero16 = jnp.zeros((16,), jnp.float32)
    one16 = jnp.ones((16,), jnp.float32)

    chkx = blob.at[pl.ds(0, _CVEC * 16)]
    chky = blob.at[pl.ds(_CVEC * 16, _CVEC * 16)]
    chkw = blob.at[pl.ds(2 * _CVEC * 16, _CVEC * 16)]
    chkh = blob.at[pl.ds(3 * _CVEC * 16, _CVEC * 16)]
    cursor = blob.at[pl.ds(_O_CURSOR, _NBH)]
    stage = blob.at[pl.ds(_O_STAGE, _NBH * _SEG)]
    cntsv = blob.at[pl.ds(_O_CNTS, _NW * _NB)]
    segbuf = blob.at[pl.ds(_O_SEGB, _NW * _SEG)]

    start_v = wid * 390 + jnp.minimum(wid, 10)
    has_tail = wid < 10

    for rnd in range(2):
        def zero_cursor(i, _):
            cursor[pl.ds(i * 16, 16)] = zero16
            return 0

        lax.fori_loop(0, _NBH // 16, zero_cursor, 0)

        def route_vec(vlocal, rnd=rnd):
            o = vlocal * 16
            x = chkx[pl.ds(o, 16)]
            y = chky[pl.ds(o, 16)]
            wi = chkw[pl.ds(o, 16)]
            hi = chkh[pl.ds(o, 16)]
            cx = (x * _W).astype(jnp.int32)
            cy = (y * _H).astype(jnp.int32)
            bucket = cy >> 3
            m = (bucket >> 7) == rnd
            bloc = bucket & (_NBH - 1)
            rel = (cy & 7) * _W + cx
            rank, lastm = plsc.scan_count(bloc, mask=m)
            basec = plsc.bitcast(plsc.load_gather(cursor, [bloc]), jnp.int32)
            pos = jnp.minimum(basec + rank - 1, _CAP - 1)
            addr = bloc * _SEG + pos * 3
            plsc.store_scatter(stage, [addr],
                               plsc.bitcast(rel, jnp.float32), mask=m)
            plsc.store_scatter(stage, [addr + 1], wi, mask=m)
            plsc.store_scatter(stage, [addr + 2], hi, mask=m)
            plsc.store_scatter(cursor, [bloc],
                               plsc.bitcast(pos + 1, jnp.float32), mask=lastm)

        def chunk_body(c, _):
            cs = start_v + c * _CVEC
            pltpu.sync_copy(xs.at[pl.ds(cs * 16, _CVEC * 16)], chkx)
            pltpu.sync_copy(ys.at[pl.ds(cs * 16, _CVEC * 16)], chky)
            pltpu.sync_copy(ws.at[pl.ds(cs * 16, _CVEC * 16)], chkw)
            pltpu.sync_copy(hs.at[pl.ds(cs * 16, _CVEC * 16)], chkh)

            def vec_body(vl, _):
                route_vec(vl)
                return 0

            lax.fori_loop(0, _CVEC, vec_body, 0)
            return 0

        lax.fori_loop(0, _NCHUNK, chunk_body, 0)

        @pl.when(has_tail)
        def _tail():
            ts = start_v + _NCHUNK * _CVEC
            pltpu.sync_copy(xs.at[pl.ds(ts * 16, 16)], blob.at[pl.ds(0, 16)])
            pltpu.sync_copy(ys.at[pl.ds(ts * 16, 16)],
                            blob.at[pl.ds(_CVEC * 16, 16)])
            pltpu.sync_copy(ws.at[pl.ds(ts * 16, 16)],
                            blob.at[pl.ds(2 * _CVEC * 16, 16)])
            pltpu.sync_copy(hs.at[pl.ds(ts * 16, 16)],
                            blob.at[pl.ds(3 * _CVEC * 16, 16)])
            route_vec(0)

        def ship(bb, _, rnd=rnd):
            b = rnd * _NBH + bb
            pltpu.async_copy(
                blob.at[pl.ds(_O_STAGE + bb * _SEG, _SEG)],
                sseg.at[pl.ds((b * _NW + wid) * _SEG, _SEG)],
                sem,
            )
            return 0

        lax.fori_loop(0, _NBH, ship, 0)

        def drain(bb, _):
            pltpu.make_async_copy(
                blob.at[pl.ds(_O_STAGE, _SEG)],
                sseg.at[pl.ds(wid * _SEG, _SEG)],
                sem,
            ).wait()
            return 0

        lax.fori_loop(0, _NBH, drain, 0)
        pltpu.sync_copy(cursor,
                        scnt.at[pl.ds(wid * _NB + rnd * _NBH, _NBH)])

    plsc.subcore_barrier()

    pltpu.sync_copy(scnt, cntsv)

    def zero_planes(i, _):
        planes[i >> 7, pl.ds((i & 127) * 16, 16)] = zero16
        return 0

    lax.fori_loop(0, 24 * 128, zero_planes, 0)

    def bucket_body(k, _):
        b = wid * 16 + k
        pltpu.sync_copy(sseg.at[pl.ds(b * _NW * _SEG, _NW * _SEG)], segbuf)
        cbvec = plsc.bitcast(
            plsc.load_gather(cntsv, [lane * _NB + b]), jnp.int32)

        def seg_pass(value_sel):
            for s in range(_NW):
                cs = jnp.sum(jnp.where(lane == s, cbvec, 0))
                nvs = (cs + 15) >> 4

                def seg_vec(v, _):
                    j = v * 16 + lane
                    idx = s * _SEG + j * 3
                    rel = plsc.bitcast(
                        plsc.load_gather(segbuf, [idx]), jnp.int32)
                    r = rel >> 11
                    c = rel & (_W - 1)
                    valm = j < cs
                    if value_sel == 0:
                        wv = plsc.load_gather(segbuf, [idx + 1])
                        hv = plsc.load_gather(segbuf, [idx + 2])
                        plsc.store_scatter(planes, [r, c], one16, mask=valm)
                        plsc.store_scatter(planes, [r + 8, c], wv, mask=valm)
                        plsc.store_scatter(planes, [r + 16, c], hv,
                                           mask=valm)
                    else:
                        plsc.store_scatter(planes, [r, c], zero16, mask=valm)
                        plsc.store_scatter(planes, [r + 8, c], zero16,
                                           mask=valm)
                        plsc.store_scatter(planes, [r + 16, c], zero16,
                                           mask=valm)
                    return 0

                lax.fori_loop(0, nvs, seg_vec, 0)

        seg_pass(0)
        r0 = b * 8
        pltpu.sync_copy(planes.at[pl.ds(0, 8), :],
                        heat.at[pl.ds(r0, 8), :])
        pltpu.sync_copy(planes.at[pl.ds(8, 8), :],
                        size.at[0, pl.ds(r0, 8), :])
        pltpu.sync_copy(planes.at[pl.ds(16, 8), :],
                        size.at[1, pl.ds(r0, 8), :])
        seg_pass(1)
        return 0

    lax.fori_loop(0, 16, bucket_body, 0)


_sc_kernel = functools.partial(
    pl.kernel,
    out_type=(
        jax.ShapeDtypeStruct((_H, _W), jnp.float32),
        jax.ShapeDtypeStruct((2, _H, _W), jnp.float32),
    ),
    mesh=plsc.VectorSubcoreMesh(
        core_axis_name="c", subcore_axis_name="s",
        num_cores=1, num_subcores=_NW,
    ),
    compiler_params=pltpu.CompilerParams(needs_layout_passes=False),
    scratch_types=[
        pltpu.VMEM((_BLOB,), jnp.float32),
        pltpu.VMEM((24, _W), jnp.float32),
        pltpu.VMEM_SHARED((_NB * _NW * _SEG,), jnp.float32),
        pltpu.VMEM_SHARED((_NW * _NB,), jnp.float32),
        pltpu.SemaphoreType.DMA,
    ],
)(_sc_body)


def kernel(boxes):
    heat, size = _sc_kernel(boxes[:, 0], boxes[:, 1], boxes[:, 2],
                            boxes[:, 3])
    return heat[None, None], size[None]


# speedup vs baseline: 1.5262x; 1.4158x over previous
import functools

import jax
import jax.numpy as jnp
from jax import lax
from jax.experimental import pallas as pl
from jax.experimental.pallas import tpu as pltpu
from jax.experimental.pallas import tpu_sc as plsc

_H = 2048
_W = 2048
_NBOX = 100000
_NW = 16
_NB = 256
_NBH = _NB // 2
_CAP = 66
_SEG = 200
_CVEC = 26
_NCHUNK = 15
_CW = _CVEC * 16
_CBUF = 4 * _CW

_O_CURSOR = 2 * _CBUF
_O_STAGE = 3456
_O_CNTS = 0
_O_SEGB = 4096
_BLOB = 29056


def _sc_body(xs, ys, ws, hs, heat, size, blob, planes, sseg, scnt,
             semc, sems, semb, semp):
    wid = lax.axis_index("s")
    lane = lax.iota(jnp.int32, 16)
    zero16 = jnp.zeros((16,), jnp.float32)
    one16 = jnp.ones((16,), jnp.float32)

    cursor = blob.at[pl.ds(_O_CURSOR, _NBH)]
    stage = blob.at[pl.ds(_O_STAGE, _NBH * _SEG)]
    cntsv = blob.at[pl.ds(_O_CNTS, _NW * _NB)]

    start_v = wid * 390 + jnp.minimum(wid, 10)
    has_tail = wid < 10
    cols = (xs, ys, ws, hs)

    def issue_chunk(c, buf):
        cs = (start_v + c * _CVEC) * 16
        for i, col in enumerate(cols):
            pltpu.async_copy(col.at[pl.ds(cs, _CW)],
                             blob.at[pl.ds(buf * _CBUF + i * _CW, _CW)],
                             semc)

    def wait_chunk():
        for i in range(4):
            pltpu.make_async_copy(xs.at[pl.ds(0, _CW)],
                                  blob.at[pl.ds(i * _CW, _CW)],
                                  semc).wait()

    for rnd in range(2):
        def zero_cursor(i, _):
            cursor[pl.ds(i * 16, 16)] = zero16
            return 0

        lax.fori_loop(0, _NBH // 16, zero_cursor, 0)

        def route_vec(vbase, rnd=rnd):
            x = blob[pl.ds(vbase, 16)]
            y = blob[pl.ds(vbase + _CW, 16)]
            wi = blob[pl.ds(vbase + 2 * _CW, 16)]
            hi = blob[pl.ds(vbase + 3 * _CW, 16)]
            cx = (x * _W).astype(jnp.int32)
            cy = (y * _H).astype(jnp.int32)
            bucket = cy >> 3
            m = (bucket >> 7) == rnd
            bloc = bucket & (_NBH - 1)
            rel = (cy & 7) * _W + cx
            rank, lastm = plsc.scan_count(bloc, mask=m)
            basec = plsc.bitcast(plsc.load_gather(cursor, [bloc]), jnp.int32)
            pos = jnp.minimum(basec + rank - 1, _CAP - 1)
            addr = bloc * _SEG + pos * 3
            plsc.store_scatter(stage, [addr],
                               plsc.bitcast(rel, jnp.float32), mask=m)
            plsc.store_scatter(stage, [addr + 1], wi, mask=m)
            plsc.store_scatter(stage, [addr + 2], hi, mask=m)
            plsc.store_scatter(cursor, [bloc],
                               plsc.bitcast(pos + 1, jnp.float32), mask=lastm)

        issue_chunk(0, 0)

        def chunk_body(c, _):
            wait_chunk()

            @pl.when(c < _NCHUNK - 1)
            def _pf():
                issue_chunk(c + 1, (c + 1) & 1)

            cbase = (c & 1) * _CBUF

            def vec_body(vl, _):
                route_vec(cbase + vl * 16)
                return 0

            lax.fori_loop(0, _CVEC, vec_body, 0)
            return 0

        lax.fori_loop(0, _NCHUNK, chunk_body, 0)

        @pl.when(has_tail)
        def _tail():
            ts = (start_v + _NCHUNK * _CVEC) * 16
            for i, col in enumerate(cols):
                pltpu.sync_copy(col.at[pl.ds(ts, 16)],
                                blob.at[pl.ds(i * _CW, 16)])
            route_vec(0)

        def ship(bb, _, rnd=rnd):
            b = rnd * _NBH + bb
            pltpu.async_copy(
                blob.at[pl.ds(_O_STAGE + bb * _SEG, _SEG)],
                sseg.at[pl.ds((b * _NW + wid) * _SEG, _SEG)],
                sems,
            )
            return 0

        lax.fori_loop(0, _NBH, ship, 0)

        def drain(bb, _):
            pltpu.make_async_copy(
                blob.at[pl.ds(_O_STAGE, _SEG)],
                sseg.at[pl.ds(wid * _SEG, _SEG)],
                sems,
            ).wait()
            return 0

        lax.fori_loop(0, _NBH, drain, 0)
        pltpu.sync_copy(cursor,
                        scnt.at[pl.ds(wid * _NB + rnd * _NBH, _NBH)])

    plsc.subcore_barrier()

    pltpu.sync_copy(scnt, cntsv)

    def zero_planes(i, _):
        planes[i >> 7, pl.ds((i & 127) * 16, 16)] = zero16
        return 0

    lax.fori_loop(0, 24 * 128, zero_planes, 0)

    def issue_seg(k, buf):
        b = wid * 16 + k
        pltpu.async_copy(sseg.at[pl.ds(b * _NW * _SEG, _NW * _SEG)],
                         blob.at[pl.ds(_O_SEGB + buf * _NW * _SEG,
                                       _NW * _SEG)],
                         semb)

    def wait_seg():
        pltpu.make_async_copy(
            sseg.at[pl.ds(0, _NW * _SEG)],
            blob.at[pl.ds(_O_SEGB, _NW * _SEG)],
            semb,
        ).wait()

    issue_seg(0, 0)

    def bucket_body(k, _):
        b = wid * 16 + k
        wait_seg()

        @pl.when(k < 15)
        def _pf():
            issue_seg(k + 1, (k + 1) & 1)

        sbase = _O_SEGB + (k & 1) * _NW * _SEG
        cbvec = plsc.bitcast(
            plsc.load_gather(cntsv, [lane * _NB + b]), jnp.int32)

        def seg_pass(value_sel):
            for s in range(_NW):
                cs = jnp.sum(jnp.where(lane == s, cbvec, 0))
                nvs = (cs + 15) >> 4

                def seg_vec(v, _):
                    j = v * 16 + lane
                    idx = sbase + s * _SEG + j * 3
                    rel = plsc.bitcast(
                        plsc.load_gather(blob, [idx]), jnp.int32)
                    r = rel >> 11
                    c = rel & (_W - 1)
                    valm = j < cs
                    if value_sel == 0:
                        wv = plsc.load_gather(blob, [idx + 1])
                        hv = plsc.load_gather(blob, [idx + 2])
                        plsc.store_scatter(planes, [r, c], one16, mask=valm)
                        plsc.store_scatter(planes, [r + 8, c], wv, mask=valm)
                        plsc.store_scatter(planes, [r + 16, c], hv,
                                           mask=valm)
                    else:
                        plsc.store_scatter(planes, [r, c], zero16, mask=valm)
                        plsc.store_scatter(planes, [r + 8, c], zero16,
                                           mask=valm)
                        plsc.store_scatter(planes, [r + 16, c], zero16,
                                           mask=valm)
                    return 0

                lax.fori_loop(0, nvs, seg_vec, 0)

        seg_pass(0)
        r0 = b * 8
        d0 = pltpu.async_copy(planes.at[pl.ds(0, 8), :],
                              heat.at[pl.ds(r0, 8), :], semp)
        d1 = pltpu.async_copy(planes.at[pl.ds(8, 8), :],
                              size.at[0, pl.ds(r0, 8), :], semp)
        d2 = pltpu.async_copy(planes.at[pl.ds(16, 8), :],
                              size.at[1, pl.ds(r0, 8), :], semp)
        d0.wait()
        d1.wait()
        d2.wait()
        seg_pass(1)
        return 0

    lax.fori_loop(0, 16, bucket_body, 0)


_sc_kernel = functools.partial(
    pl.kernel,
    out_type=(
        jax.ShapeDtypeStruct((_H, _W), jnp.float32),
        jax.ShapeDtypeStruct((2, _H, _W), jnp.float32),
    ),
    mesh=plsc.VectorSubcoreMesh(
        core_axis_name="c", subcore_axis_name="s",
        num_cores=1, num_subcores=_NW,
    ),
    compiler_params=pltpu.CompilerParams(needs_layout_passes=False),
    scratch_types=[
        pltpu.VMEM((_BLOB,), jnp.float32),
        pltpu.VMEM((24, _W), jnp.float32),
        pltpu.VMEM_SHARED((_NB * _NW * _SEG,), jnp.float32),
        pltpu.VMEM_SHARED((_NW * _NB,), jnp.float32),
        pltpu.SemaphoreType.DMA,
        pltpu.SemaphoreType.DMA,
        pltpu.SemaphoreType.DMA,
        pltpu.SemaphoreType.DMA,
    ],
)(_sc_body)


def kernel(boxes):
    heat, size = _sc_kernel(boxes[:, 0], boxes[:, 1], boxes[:, 2],
                            boxes[:, 3])
    return heat[None, None], size[None]
